# baseline (speedup 1.0000x reference)
import functools

import jax
import jax.numpy as jnp
from jax import lax
from jax.experimental import pallas as pl
from jax.experimental.pallas import tpu as pltpu
from jax.experimental.pallas import tpu_sc as plsc

VOCAB = 1000000
EMBED_DIM = 32
BATCH = 16384
HIST = 50
TOTAL = BATCH * HIST

_NUM_WORKERS = 32
_STRIPE = BATCH // _NUM_WORKERS

_mesh = plsc.VectorSubcoreMesh(core_axis_name="c", subcore_axis_name="s")


@functools.partial(
    pl.kernel,
    mesh=_mesh,
    out_type=jax.ShapeDtypeStruct((HIST, EMBED_DIM, BATCH), jnp.float32),
    scratch_types=[
        pltpu.VMEM((2, _STRIPE), jnp.int32),
        pltpu.VMEM((2, _STRIPE, EMBED_DIM), jnp.float32),
        pltpu.VMEM((2, EMBED_DIM, _STRIPE), jnp.float32),
        pltpu.SemaphoreType.DMA((2,)),
        pltpu.SemaphoreType.DMA((2,)),
    ],
    compiler_params=pltpu.CompilerParams(use_tc_tiling_on_sc=False,
                                         needs_layout_passes=False),
)
def _gather_kernel(idx_hbm, table_hbm, out_hbm, idx_v, rows_v, trows_v,
                   gsems, osems):
    wid = lax.axis_index("s") * 2 + lax.axis_index("c")
    col0 = wid * _STRIPE
    iota16 = lax.iota(jnp.int32, 16)
    ecols = [jnp.full((16,), e, jnp.int32) for e in range(EMBED_DIM)]

    def start_gather(h, b):
        pltpu.sync_copy(idx_hbm.at[pl.ds(h * BATCH + col0, _STRIPE)],
                        idx_v.at[b])
        pltpu.make_async_copy(table_hbm.at[idx_v.at[b]], rows_v.at[b],
                              gsems.at[b]).start()

    def out_copy(h, b):
        return pltpu.make_async_copy(
            trows_v.at[b],
            out_hbm.at[h, :, pl.ds(col0, _STRIPE)],
            osems.at[b])

    def transpose_block(b):
        @plsc.parallel_loop(0, _STRIPE // 16, 1, unroll=8)
        def jbody(j):
            rid = iota16 + j * 16
            for e in range(EMBED_DIM):
                v = plsc.load_gather(rows_v.at[b], [rid, ecols[e]])
                trows_v.at[b][e, pl.ds(j * 16, 16)] = v

    start_gather(0, 0)

    def slot(h, b):
        @pl.when(h + 1 < HIST)
        def _():
            start_gather(h + 1, 1 - b)
        pltpu.make_async_copy(table_hbm.at[idx_v.at[b]], rows_v.at[b],
                              gsems.at[b]).wait()

        @pl.when(h >= 2)
        def _():
            out_copy(h - 2, b).wait()
        transpose_block(b)
        out_copy(h, b).start()

    def gbody(g, carry):
        slot(2 * g, 0)
        slot(2 * g + 1, 1)
        return carry

    lax.fori_loop(0, HIST // 2, gbody, 0)
    out_copy(HIST - 2, 0).wait()
    out_copy(HIST - 1, 1).wait()


def kernel(x, vectors):
    idx = x.T.reshape(-1).astype(jnp.int32)
    out = _gather_kernel(idx, vectors)
    return out.transpose(2, 0, 1)


# speedup vs baseline: 1.1363x; 1.1363x over previous
import functools

import jax
import jax.numpy as jnp
from jax import lax
from jax.experimental import pallas as pl
from jax.experimental.pallas import tpu as pltpu
from jax.experimental.pallas import tpu_sc as plsc

VOCAB = 1000000
EMBED_DIM = 32
BATCH = 16384
HIST = 50
TOTAL = BATCH * HIST

_NUM_WORKERS = 32
_PER_WORKER = TOTAL // _NUM_WORKERS
_CHUNK = 1280
_NCHUNKS = _PER_WORKER // _CHUNK
_NBUF = 3

_mesh = plsc.VectorSubcoreMesh(core_axis_name="c", subcore_axis_name="s")


@functools.partial(
    pl.kernel,
    mesh=_mesh,
    out_type=jax.ShapeDtypeStruct((TOTAL, EMBED_DIM), jnp.float32),
    scratch_types=[
        pltpu.VMEM((_NBUF, _CHUNK), jnp.int32),
        pltpu.VMEM((_NBUF, _CHUNK, EMBED_DIM), jnp.float32),
        pltpu.SemaphoreType.DMA((_NBUF,)),
        pltpu.SemaphoreType.DMA((_NBUF,)),
    ],
    compiler_params=pltpu.CompilerParams(use_tc_tiling_on_sc=False),
)
def _gather_kernel(idx_hbm, table_hbm, out_hbm, idx_v, rows_v, gsems, osems):
    wid = lax.axis_index("s") * 2 + lax.axis_index("c")
    base0 = wid * _PER_WORKER

    def start_chunk(j):
        b = j % _NBUF
        pltpu.sync_copy(idx_hbm.at[pl.ds(base0 + j * _CHUNK, _CHUNK)],
                        idx_v.at[b])
        pltpu.make_async_copy(table_hbm.at[idx_v.at[b]], rows_v.at[b],
                              gsems.at[b]).start()

    def out_copy(i):
        b = i % _NBUF
        return pltpu.make_async_copy(
            rows_v.at[b],
            out_hbm.at[pl.ds(base0 + i * _CHUNK, _CHUNK)],
            osems.at[b])

    for j in range(_NBUF - 1):
        start_chunk(j)

    for i in range(_NCHUNKS):
        b = i % _NBUF
        j = i + _NBUF - 1
        if j < _NCHUNKS:
            if j - _NBUF >= 0:
                out_copy(j - _NBUF).wait()
            start_chunk(j)
        pltpu.make_async_copy(table_hbm.at[idx_v.at[b]], rows_v.at[b],
                              gsems.at[b]).wait()
        out_copy(i).start()

    for i in range(max(0, _NCHUNKS - _NBUF), _NCHUNKS):
        out_copy(i).wait()


def kernel(x, vectors):
    idx = x.T.reshape(-1).astype(jnp.int32)
    out = _gather_kernel(idx, vectors)
    return out.reshape(HIST, BATCH, EMBED_DIM).transpose(1, 0, 2)
